# baseline (speedup 1.0000x reference)
import functools

import jax
import jax.numpy as jnp
from jax import lax
from jax.experimental import pallas as pl
from jax.experimental.pallas import tpu as pltpu
from jax.experimental.pallas import tpu_sc as plsc

B = 128
V = 100000
C = 0.1
NWORK = B // 16

TC_W = 55040
SC_COL0 = TC_W
SC_PER_CORE = 22400
SC_WIDTHS = (6400, 6400, 6400, 3200)
SC_ITERS = (50, 50, 50, 25)
STRIP0 = 99840
STRIP1 = 99968

RG = 8
K = B // RG
NTILE = 32


def _sc_gather_body(target_hbm, pred_hbm, out_hbm, tgt_v, vals_v, diag_v, sem):
    wid = lax.axis_index("s") * 2 + lax.axis_index("c")

    @pl.when(wid < NWORK)
    def _():
        base = wid * 16
        pltpu.sync_copy(target_hbm.at[pl.ds(base, 16)], tgt_v)
        tv = tgt_v[...]
        handles = []
        offs = []
        for i in range(16):
            t = tv[i]
            cs = (t // 16) * 16
            handles.append(
                pltpu.async_copy(
                    pred_hbm.at[base + i, pl.ds(cs, 16)], vals_v.at[i], sem
                )
            )
            offs.append(t - cs)
        for h in handles:
            h.wait()
        ii = lax.iota(jnp.int32, 16)
        d = jnp.full((16,), -C, jnp.float32)
        for i in range(16):
            off = jnp.full((16,), offs[i], jnp.int32)
            g = vals_v[i, :].at[off].get(mode="promise_in_bounds")
            d = jnp.where(ii == i, g - C, d)
        diag_v[...] = d
        pltpu.sync_copy(diag_v, out_hbm.at[pl.ds(base, 16)])


_sc_gather = functools.partial(
    pl.kernel,
    mesh=plsc.VectorSubcoreMesh(core_axis_name="c", subcore_axis_name="s"),
    out_type=jax.ShapeDtypeStruct((B,), jnp.float32),
    scratch_types=[
        pltpu.VMEM((16,), jnp.int32),
        pltpu.VMEM((16, 16), jnp.float32),
        pltpu.VMEM((16,), jnp.float32),
        pltpu.SemaphoreType.DMA,
    ],
)(_sc_gather_body)


def _relu_sum_rows(buf_v, corrc, acc, iters):
    for r in range(8):
        cc = corrc[r]

        def body(i, a, r=r, cc=cc):
            base = i * 128
            for u in range(8):
                x = buf_v[r, pl.ds(base + 16 * u, 16)]
                a = a + jnp.maximum(x - cc, 0.0)
            return a

        acc = lax.fori_loop(0, iters, body, acc)
    return acc


def _sc_stream_body(target_hbm, pred_hbm, out_hbm,
                    tgt_v, win_v, buf0_v, buf1_v, part_v, sem0, sem1, gsem):
    s = lax.axis_index("s")
    c = lax.axis_index("c")
    wid = s * 2 + c
    row0 = pl.multiple_of(8 * s, 8)
    col0 = pl.multiple_of(SC_COL0 + c * SC_PER_CORE, 128)

    tstart = jnp.minimum(row0, B - 16)
    pltpu.sync_copy(target_hbm.at[pl.ds(tstart, 16)], tgt_v)
    ii = lax.iota(jnp.int32, 16)
    pos = jnp.minimum((row0 - tstart) + ii, 15)
    tvv = tgt_v[...].at[pos].get(mode="promise_in_bounds")
    handles = []
    offs = []
    for r in range(8):
        t = tvv[r]
        cs = (t // 16) * 16
        handles.append(
            pltpu.async_copy(pred_hbm.at[row0 + r, pl.ds(cs, 16)],
                             win_v.at[r], gsem)
        )
        offs.append(t - cs)
    for h in handles:
        h.wait()
    corrc = []
    for r in range(8):
        off = jnp.full((16,), offs[r], jnp.int32)
        g = win_v[r, :].at[off].get(mode="promise_in_bounds")
        corrc.append(g - C)

    bufs = [buf0_v, buf1_v]
    sems = [sem0, sem1]
    starts = []
    o = 0
    for w in SC_WIDTHS:
        starts.append(o)
        o += w

    def issue(k):
        w = SC_WIDTHS[k]
        return pltpu.async_copy(
            pred_hbm.at[pl.ds(row0, 8),
                        pl.ds(pl.multiple_of(col0 + starts[k], 128), w)],
            bufs[k % 2].at[:, pl.ds(0, w)],
            sems[k % 2],
        )

    acc = jnp.zeros((16,), jnp.float32)
    h_cur = issue(0)
    n = len(SC_WIDTHS)
    for k in range(n):
        h_next = issue(k + 1) if k + 1 < n else None
        h_cur.wait()
        acc = _relu_sum_rows(bufs[k % 2], corrc, acc, SC_ITERS[k])
        h_cur = h_next

    part_v[...] = acc
    pltpu.sync_copy(part_v, out_hbm.at[pl.ds(wid * 16, 16)])


_sc_stream = functools.partial(
    pl.kernel,
    mesh=plsc.VectorSubcoreMesh(core_axis_name="c", subcore_axis_name="s"),
    out_type=jax.ShapeDtypeStruct((NTILE * 16,), jnp.float32),
    scratch_types=[
        pltpu.VMEM((16,), jnp.int32),
        pltpu.VMEM((8, 16), jnp.float32),
        pltpu.VMEM((8, 6400), jnp.float32),
        pltpu.VMEM((8, 6400), jnp.float32),
        pltpu.VMEM((16,), jnp.float32),
        pltpu.SemaphoreType.DMA,
        pltpu.SemaphoreType.DMA,
        pltpu.SemaphoreType.DMA,
    ],
)(_sc_stream_body)


def _tc_body(corrc_ref, pred_ref, out_ref, acc_ref):
    k = pl.program_id(0)
    s = jnp.sum(jnp.maximum(pred_ref[...] - corrc_ref[...], 0.0))

    @pl.when(k == 0)
    def _():
        acc_ref[0] = s

    @pl.when(k > 0)
    def _():
        acc_ref[0] += s

    @pl.when(k == K - 1)
    def _():
        out_ref[0] = acc_ref[0]


def _final_body(tcp_ref, part_ref, corr_ref, pa_ref, pb_ref, out_ref):
    corr = corr_ref[...]
    sa = jnp.sum(jnp.maximum(pa_ref[...] - corr, 0.0))
    lanes = lax.broadcasted_iota(jnp.int32, (B, 128), 1)
    tb = jnp.maximum(pb_ref[...] - corr, 0.0)
    sb = jnp.sum(jnp.where(lanes < V - STRIP1, tb, 0.0))
    out_ref[0] = (tcp_ref[0] + jnp.sum(part_ref[...]) + sa + sb - B * C) / B


def kernel(target, prediction):
    target = target.astype(jnp.int32)
    partials = _sc_stream(target, prediction)
    corrc = _sc_gather(target, prediction)
    tc_part = pl.pallas_call(
        _tc_body,
        grid=(K,),
        in_specs=[
            pl.BlockSpec((RG, 1), lambda k: (k, 0)),
            pl.BlockSpec((RG, TC_W), lambda k: (k, 0)),
        ],
        out_specs=pl.BlockSpec(memory_space=pltpu.SMEM),
        out_shape=jax.ShapeDtypeStruct((1,), jnp.float32),
        scratch_shapes=[pltpu.SMEM((1,), jnp.float32)],
    )(corrc.reshape(B, 1), prediction)
    out = pl.pallas_call(
        _final_body,
        grid=(1,),
        in_specs=[
            pl.BlockSpec(memory_space=pltpu.SMEM),
            pl.BlockSpec((NTILE, 16), lambda k: (0, 0)),
            pl.BlockSpec((B, 1), lambda k: (0, 0)),
            pl.BlockSpec((B, 128), lambda k: (0, STRIP0 // 128)),
            pl.BlockSpec((B, 128), lambda k: (0, STRIP1 // 128)),
        ],
        out_specs=pl.BlockSpec(memory_space=pltpu.SMEM),
        out_shape=jax.ShapeDtypeStruct((1,), jnp.float32),
    )(tc_part, partials.reshape(NTILE, 16), corrc.reshape(B, 1),
      prediction, prediction)
    return out


# speedup vs baseline: 1.0011x; 1.0011x over previous
import functools

import jax
import jax.numpy as jnp
from jax import lax
from jax.experimental import pallas as pl
from jax.experimental.pallas import tpu as pltpu
from jax.experimental.pallas import tpu_sc as plsc

B = 128
V = 100000
C = 0.1
NWORK = B // 16

TC_W = 55040
SC_COL0 = TC_W
SC_PER_CORE = 22400
SC_WIDTHS = (6400, 6400, 6400, 3200)
SC_ITERS = (50, 50, 50, 25)
STRIP0 = 99840
STRIP1 = 99968

RG = 8
K = B // RG
NTILE = 32


def _sc_gather_body(target_hbm, pred_hbm, out_hbm, tgt_v, vals_v, diag_v, sem):
    wid = lax.axis_index("s") * 2 + lax.axis_index("c")

    @pl.when(wid < NWORK)
    def _():
        base = wid * 16
        pltpu.sync_copy(target_hbm.at[pl.ds(base, 16)], tgt_v)
        tv = tgt_v[...]
        handles = []
        offs = []
        for i in range(16):
            t = tv[i]
            cs = (t // 16) * 16
            handles.append(
                pltpu.async_copy(
                    pred_hbm.at[base + i, pl.ds(cs, 16)], vals_v.at[i], sem
                )
            )
            offs.append(t - cs)
        for h in handles:
            h.wait()
        ii = lax.iota(jnp.int32, 16)
        d = jnp.full((16,), -C, jnp.float32)
        for i in range(16):
            off = jnp.full((16,), offs[i], jnp.int32)
            g = vals_v[i, :].at[off].get(mode="promise_in_bounds")
            d = jnp.where(ii == i, g - C, d)
        diag_v[...] = d
        pltpu.sync_copy(diag_v, out_hbm.at[pl.ds(base, 16)])


_sc_gather = functools.partial(
    pl.kernel,
    mesh=plsc.VectorSubcoreMesh(core_axis_name="c", subcore_axis_name="s"),
    out_type=jax.ShapeDtypeStruct((B,), jnp.float32),
    scratch_types=[
        pltpu.VMEM((16,), jnp.int32),
        pltpu.VMEM((16, 16), jnp.float32),
        pltpu.VMEM((16,), jnp.float32),
        pltpu.SemaphoreType.DMA,
    ],
    compiler_params=pltpu.CompilerParams(use_tc_tiling_on_sc=True),
)(_sc_gather_body)


def _relu_sum_rows(buf_v, corrc, acc, iters):
    for r in range(8):
        cc = corrc[r]

        def body(i, a, r=r, cc=cc):
            base = i * 128
            for u in range(8):
                x = buf_v[r, pl.ds(base + 16 * u, 16)]
                a = a + jnp.maximum(x - cc, 0.0)
            return a

        acc = lax.fori_loop(0, iters, body, acc)
    return acc


def _sc_stream_body(target_hbm, pred_hbm, out_hbm,
                    tgt_v, win_v, buf0_v, buf1_v, part_v, sem0, sem1, gsem):
    s = lax.axis_index("s")
    c = lax.axis_index("c")
    wid = s * 2 + c
    row0 = pl.multiple_of(8 * s, 8)
    col0 = pl.multiple_of(SC_COL0 + c * SC_PER_CORE, 128)

    tstart = jnp.minimum(row0, B - 16)
    pltpu.sync_copy(target_hbm.at[pl.ds(tstart, 16)], tgt_v)
    ii = lax.iota(jnp.int32, 16)
    pos = jnp.minimum((row0 - tstart) + ii, 15)
    tvv = tgt_v[...].at[pos].get(mode="promise_in_bounds")
    handles = []
    offs = []
    for r in range(8):
        t = tvv[r]
        cs = (t // 16) * 16
        handles.append(
            pltpu.async_copy(pred_hbm.at[row0 + r, pl.ds(cs, 16)],
                             win_v.at[r], gsem)
        )
        offs.append(t - cs)
    for h in handles:
        h.wait()
    corrc = []
    for r in range(8):
        off = jnp.full((16,), offs[r], jnp.int32)
        g = win_v[r, :].at[off].get(mode="promise_in_bounds")
        corrc.append(g - C)

    bufs = [buf0_v, buf1_v]
    sems = [sem0, sem1]
    starts = []
    o = 0
    for w in SC_WIDTHS:
        starts.append(o)
        o += w

    def issue(k):
        w = SC_WIDTHS[k]
        return pltpu.async_copy(
            pred_hbm.at[pl.ds(row0, 8),
                        pl.ds(pl.multiple_of(col0 + starts[k], 128), w)],
            bufs[k % 2].at[:, pl.ds(0, w)],
            sems[k % 2],
        )

    acc = jnp.zeros((16,), jnp.float32)
    h_cur = issue(0)
    n = len(SC_WIDTHS)
    for k in range(n):
        h_next = issue(k + 1) if k + 1 < n else None
        h_cur.wait()
        acc = _relu_sum_rows(bufs[k % 2], corrc, acc, SC_ITERS[k])
        h_cur = h_next

    part_v[...] = acc
    pltpu.sync_copy(part_v, out_hbm.at[pl.ds(wid * 16, 16)])


_sc_stream = functools.partial(
    pl.kernel,
    mesh=plsc.VectorSubcoreMesh(core_axis_name="c", subcore_axis_name="s"),
    out_type=jax.ShapeDtypeStruct((NTILE * 16,), jnp.float32),
    scratch_types=[
        pltpu.VMEM((16,), jnp.int32),
        pltpu.VMEM((8, 16), jnp.float32),
        pltpu.VMEM((8, 6400), jnp.float32),
        pltpu.VMEM((8, 6400), jnp.float32),
        pltpu.VMEM((16,), jnp.float32),
        pltpu.SemaphoreType.DMA,
        pltpu.SemaphoreType.DMA,
        pltpu.SemaphoreType.DMA,
    ],
    compiler_params=pltpu.CompilerParams(use_tc_tiling_on_sc=True),
)(_sc_stream_body)


def _tc_body(corrc_ref, pred_ref, out_ref, acc_ref):
    k = pl.program_id(0)
    s = jnp.sum(jnp.maximum(pred_ref[...] - corrc_ref[...], 0.0))

    @pl.when(k == 0)
    def _():
        acc_ref[0] = s

    @pl.when(k > 0)
    def _():
        acc_ref[0] += s

    @pl.when(k == K - 1)
    def _():
        out_ref[0] = acc_ref[0]


def _final_body(tcp_ref, part_ref, corr_ref, pa_ref, pb_ref, out_ref):
    corr = corr_ref[...]
    sa = jnp.sum(jnp.maximum(pa_ref[...] - corr, 0.0))
    lanes = lax.broadcasted_iota(jnp.int32, (B, 128), 1)
    tb = jnp.maximum(pb_ref[...] - corr, 0.0)
    sb = jnp.sum(jnp.where(lanes < V - STRIP1, tb, 0.0))
    out_ref[0] = (tcp_ref[0] + jnp.sum(part_ref[...]) + sa + sb - B * C) / B


def kernel(target, prediction):
    target = target.astype(jnp.int32)
    partials = _sc_stream(target, prediction)
    corrc = _sc_gather(target, prediction)
    tc_part = pl.pallas_call(
        _tc_body,
        grid=(K,),
        in_specs=[
            pl.BlockSpec((RG, 1), lambda k: (k, 0)),
            pl.BlockSpec((RG, TC_W), lambda k: (k, 0)),
        ],
        out_specs=pl.BlockSpec(memory_space=pltpu.SMEM),
        out_shape=jax.ShapeDtypeStruct((1,), jnp.float32),
        scratch_shapes=[pltpu.SMEM((1,), jnp.float32)],
    )(corrc.reshape(B, 1), prediction)
    out = pl.pallas_call(
        _final_body,
        grid=(1,),
        in_specs=[
            pl.BlockSpec(memory_space=pltpu.SMEM),
            pl.BlockSpec((NTILE, 16), lambda k: (0, 0)),
            pl.BlockSpec((B, 1), lambda k: (0, 0)),
            pl.BlockSpec((B, 128), lambda k: (0, STRIP0 // 128)),
            pl.BlockSpec((B, 128), lambda k: (0, STRIP1 // 128)),
        ],
        out_specs=pl.BlockSpec(memory_space=pltpu.SMEM),
        out_shape=jax.ShapeDtypeStruct((1,), jnp.float32),
    )(tc_part, partials.reshape(NTILE, 16), corrc.reshape(B, 1),
      prediction, prediction)
    return out


# speedup vs baseline: 2.0031x; 2.0008x over previous
import functools

import jax
import jax.numpy as jnp
from jax import lax
from jax.experimental import pallas as pl
from jax.experimental.pallas import tpu as pltpu
from jax.experimental.pallas import tpu_sc as plsc

B = 128
V = 100000
C = 0.1
VB = 5000
K = V // VB
NWORK = B // 16


def _sc_gather_body(target_hbm, predt_hbm, out_hbm, tgt_v, rows_v, diag_v, sem):
    wid = lax.axis_index("s") * 2 + lax.axis_index("c")

    @pl.when(wid < NWORK)
    def _():
        base = wid * 16
        pltpu.sync_copy(target_hbm.at[pl.ds(base, 16)], tgt_v)
        pltpu.async_copy(predt_hbm.at[tgt_v], rows_v, sem).wait()
        ii = lax.iota(jnp.int32, 16)
        d = jnp.full((16,), -C, jnp.float32)
        for l in range(16):
            vec = rows_v[l, pl.ds(base, 16)]
            d = jnp.where(ii == l, vec - C, d)
        diag_v[...] = d
        pltpu.sync_copy(diag_v, out_hbm.at[pl.ds(base, 16)])


_sc_gather = functools.partial(
    pl.kernel,
    mesh=plsc.VectorSubcoreMesh(core_axis_name="c", subcore_axis_name="s"),
    out_type=jax.ShapeDtypeStruct((B,), jnp.float32),
    scratch_types=[
        pltpu.VMEM((16,), jnp.int32),
        pltpu.VMEM((16, B), jnp.float32),
        pltpu.VMEM((16,), jnp.float32),
        pltpu.SemaphoreType.DMA,
    ],
    compiler_params=pltpu.CompilerParams(use_tc_tiling_on_sc=True),
)(_sc_gather_body)


def _tc_body(corrc_ref, pred_ref, out_ref, acc_ref):
    k = pl.program_id(0)
    s = jnp.sum(jnp.maximum(pred_ref[...] - corrc_ref[...], 0.0))

    @pl.when(k == 0)
    def _():
        acc_ref[0] = s

    @pl.when(k > 0)
    def _():
        acc_ref[0] += s

    @pl.when(k == K - 1)
    def _():
        out_ref[0] = (acc_ref[0] - B * C) / B


def kernel(target, prediction):
    target = target.astype(jnp.int32)
    predt = prediction.T
    corrc = _sc_gather(target, predt)
    out = pl.pallas_call(
        _tc_body,
        grid=(K,),
        in_specs=[
            pl.BlockSpec((1, B), lambda k: (0, 0)),
            pl.BlockSpec((VB, B), lambda k: (k, 0)),
        ],
        out_specs=pl.BlockSpec(memory_space=pltpu.SMEM),
        out_shape=jax.ShapeDtypeStruct((1,), jnp.float32),
        scratch_shapes=[pltpu.SMEM((1,), jnp.float32)],
    )(corrc.reshape(1, B), predt)
    return out


# speedup vs baseline: 2.2690x; 1.1327x over previous
import functools

import jax
import jax.numpy as jnp
from jax import lax
from jax.experimental import pallas as pl
from jax.experimental.pallas import tpu as pltpu
from jax.experimental.pallas import tpu_sc as plsc

B = 128
V = 100000
C = 0.1
VB = 10000
K = V // VB
NWORK = B // 16


def _sc_gather_body(target_hbm, predt_hbm, out_hbm, tgt_v, rows_v, diag_v, sem):
    wid = lax.axis_index("s") * 2 + lax.axis_index("c")

    @pl.when(wid < NWORK)
    def _():
        base = wid * 16
        pltpu.sync_copy(target_hbm.at[pl.ds(base, 16)], tgt_v)
        pltpu.async_copy(predt_hbm.at[tgt_v], rows_v, sem).wait()
        ii = lax.iota(jnp.int32, 16)
        d = jnp.full((16,), -C, jnp.float32)
        for l in range(16):
            vec = rows_v[l, pl.ds(base, 16)]
            d = jnp.where(ii == l, vec - C, d)
        diag_v[...] = d
        pltpu.sync_copy(diag_v, out_hbm.at[pl.ds(base, 16)])


_sc_gather = functools.partial(
    pl.kernel,
    mesh=plsc.VectorSubcoreMesh(core_axis_name="c", subcore_axis_name="s"),
    out_type=jax.ShapeDtypeStruct((B,), jnp.float32),
    scratch_types=[
        pltpu.VMEM((16,), jnp.int32),
        pltpu.VMEM((16, B), jnp.float32),
        pltpu.VMEM((16,), jnp.float32),
        pltpu.SemaphoreType.DMA,
    ],
    compiler_params=pltpu.CompilerParams(use_tc_tiling_on_sc=True),
)(_sc_gather_body)


def _tc_body(corrc_ref, pred_ref, out_ref, acc_ref):
    k = pl.program_id(0)
    s = jnp.sum(jnp.maximum(pred_ref[...] - corrc_ref[...], 0.0))

    @pl.when(k == 0)
    def _():
        acc_ref[0] = s

    @pl.when(k > 0)
    def _():
        acc_ref[0] += s

    @pl.when(k == K - 1)
    def _():
        out_ref[0] = (acc_ref[0] - B * C) / B


def kernel(target, prediction):
    target = target.astype(jnp.int32)
    predt = prediction.T
    corrc = _sc_gather(target, predt)
    out = pl.pallas_call(
        _tc_body,
        grid=(K,),
        in_specs=[
            pl.BlockSpec((1, B), lambda k: (0, 0)),
            pl.BlockSpec((VB, B), lambda k: (k, 0)),
        ],
        out_specs=pl.BlockSpec(memory_space=pltpu.SMEM),
        out_shape=jax.ShapeDtypeStruct((1,), jnp.float32),
        scratch_shapes=[pltpu.SMEM((1,), jnp.float32)],
    )(corrc.reshape(1, B), predt)
    return out


# speedup vs baseline: 2.3701x; 1.0446x over previous
import functools

import jax
import jax.numpy as jnp
from jax import lax
from jax.experimental import pallas as pl
from jax.experimental.pallas import tpu as pltpu
from jax.experimental.pallas import tpu_sc as plsc

B = 128
V = 100000
C = 0.1
VB = 25000
K = V // VB
NWORK = B // 16


def _sc_gather_body(target_hbm, predt_hbm, out_hbm, tgt_v, rows_v, diag_v, sem):
    wid = lax.axis_index("s") * 2 + lax.axis_index("c")

    @pl.when(wid < NWORK)
    def _():
        base = wid * 16
        pltpu.sync_copy(target_hbm.at[pl.ds(base, 16)], tgt_v)
        pltpu.async_copy(predt_hbm.at[tgt_v], rows_v, sem).wait()
        ii = lax.iota(jnp.int32, 16)
        d = jnp.full((16,), -C, jnp.float32)
        for l in range(16):
            vec = rows_v[l, pl.ds(base, 16)]
            d = jnp.where(ii == l, vec - C, d)
        diag_v[...] = d
        pltpu.sync_copy(diag_v, out_hbm.at[pl.ds(base, 16)])


_sc_gather = functools.partial(
    pl.kernel,
    mesh=plsc.VectorSubcoreMesh(core_axis_name="c", subcore_axis_name="s"),
    out_type=jax.ShapeDtypeStruct((B,), jnp.float32),
    scratch_types=[
        pltpu.VMEM((16,), jnp.int32),
        pltpu.VMEM((16, B), jnp.float32),
        pltpu.VMEM((16,), jnp.float32),
        pltpu.SemaphoreType.DMA,
    ],
    compiler_params=pltpu.CompilerParams(use_tc_tiling_on_sc=True),
)(_sc_gather_body)


def _tc_body(corrc_ref, pred_ref, out_ref, acc_ref):
    k = pl.program_id(0)
    s = jnp.sum(jnp.maximum(pred_ref[...] - corrc_ref[...], 0.0))

    @pl.when(k == 0)
    def _():
        acc_ref[0] = s

    @pl.when(k > 0)
    def _():
        acc_ref[0] += s

    @pl.when(k == K - 1)
    def _():
        out_ref[0] = (acc_ref[0] - B * C) / B


def kernel(target, prediction):
    target = target.astype(jnp.int32)
    predt = prediction.T
    corrc = _sc_gather(target, predt)
    out = pl.pallas_call(
        _tc_body,
        grid=(K,),
        in_specs=[
            pl.BlockSpec((1, B), lambda k: (0, 0)),
            pl.BlockSpec((VB, B), lambda k: (k, 0)),
        ],
        out_specs=pl.BlockSpec(memory_space=pltpu.SMEM),
        out_shape=jax.ShapeDtypeStruct((1,), jnp.float32),
        scratch_shapes=[pltpu.SMEM((1,), jnp.float32)],
    )(corrc.reshape(1, B), predt)
    return out
